# baseline (speedup 1.0000x reference)
import functools

import jax
import jax.numpy as jnp
from jax import lax
from jax.experimental import pallas as pl
from jax.experimental.pallas import tpu as pltpu
from jax.experimental.pallas import tpu_sc as plsc

B = 16384
D = 128
NC = 2
NS = 16
NW = NC * NS
ROWS_PER_W = B // NW
CHUNK = 128
NCHUNK = ROWS_PER_W // CHUNK
GROUPS = CHUNK // 16


def _body(gu_hbm, gi_hbm, out_hbm, gu_v0, gu_v1, gi_v0, gi_v1, out_v,
          sem_u0, sem_u1, sem_i0, sem_i1):
    wid = lax.axis_index("s") * NC + lax.axis_index("c")
    base = wid * ROWS_PER_W
    gu_bufs = (gu_v0, gu_v1)
    gi_bufs = (gi_v0, gi_v1)
    sems_u = (sem_u0, sem_u1)
    sems_i = (sem_i0, sem_i1)

    def start(c, buf):
        r0 = base + c * CHUNK
        cu = pltpu.make_async_copy(
            gu_hbm.at[pl.ds(r0, CHUNK)], gu_bufs[buf], sems_u[buf])
        ci = pltpu.make_async_copy(
            gi_hbm.at[pl.ds(r0, CHUNK)], gi_bufs[buf], sems_i[buf])
        cu.start()
        ci.start()
        return cu, ci

    pending = start(0, 0)
    for c in range(NCHUNK):
        buf = c % 2
        cu, ci = pending
        if c + 1 < NCHUNK:
            pending = start(c + 1, 1 - buf)
        cu.wait()
        ci.wait()
        gu_b = gu_bufs[buf]
        gi_b = gi_bufs[buf]

        lane = lax.iota(jnp.int32, 16)

        @plsc.parallel_loop(0, GROUPS, step=1, unroll=2)
        def group_body(g):
            vs = []
            for rr in range(16):
                r = g * 16 + rr
                acc = gu_b[r, pl.ds(0, 16)] * gi_b[r, pl.ds(0, 16)]
                for j in range(1, D // 16):
                    acc = acc + (gu_b[r, pl.ds(j * 16, 16)]
                                 * gi_b[r, pl.ds(j * 16, 16)])
                vs.append(acc)
            for k in range(4):
                bit = 1 << k
                mask = (lane & bit) == 0
                perm = lane ^ bit
                vs = [jnp.where(mask, x, y) + jnp.where(mask, y, x)[perm]
                      for x, y in zip(vs[0::2], vs[1::2])]
            out_v[pl.ds(c * CHUNK + g * 16, 16)] = vs[0]

    pltpu.sync_copy(out_v, out_hbm.at[pl.ds(base, ROWS_PER_W)])


@jax.jit
def kernel(gu, gi):
    mesh = plsc.VectorSubcoreMesh(core_axis_name="c", subcore_axis_name="s")
    f = functools.partial(
        pl.kernel,
        out_type=jax.ShapeDtypeStruct((B,), jnp.float32),
        mesh=mesh,
        scratch_types=[
            pltpu.VMEM((CHUNK, D), jnp.float32),
            pltpu.VMEM((CHUNK, D), jnp.float32),
            pltpu.VMEM((CHUNK, D), jnp.float32),
            pltpu.VMEM((CHUNK, D), jnp.float32),
            pltpu.VMEM((ROWS_PER_W,), jnp.float32),
            pltpu.SemaphoreType.DMA,
            pltpu.SemaphoreType.DMA,
            pltpu.SemaphoreType.DMA,
            pltpu.SemaphoreType.DMA,
        ],
    )(_body)
    return f(gu, gi)


# speedup vs baseline: 3.8609x; 3.8609x over previous
import jax
import jax.numpy as jnp
from jax.experimental import pallas as pl

B = 16384
D = 128
BLOCK = 2048


def _dot_body(gu_ref, gi_ref, out_ref):
    out_ref[...] = jnp.sum(gu_ref[...] * gi_ref[...], axis=-1)


@jax.jit
def kernel(gu, gi):
    return pl.pallas_call(
        _dot_body,
        grid=(B // BLOCK,),
        in_specs=[
            pl.BlockSpec((BLOCK, D), lambda i: (i, 0)),
            pl.BlockSpec((BLOCK, D), lambda i: (i, 0)),
        ],
        out_specs=pl.BlockSpec((BLOCK,), lambda i: (i,)),
        out_shape=jax.ShapeDtypeStruct((B,), jnp.float32),
    )(gu, gi)


# speedup vs baseline: 4.9714x; 1.2876x over previous
import jax
import jax.numpy as jnp
from jax.experimental import pallas as pl

B = 16384
D = 128
TB = 16


def _dot_body(gu_ref, gi_ref, out_ref):
    prod = gu_ref[...] * gi_ref[...]
    pt = jnp.swapaxes(prod, 1, 2)
    out_ref[...] = jnp.sum(pt, axis=1)


@jax.jit
def kernel(gu, gi):
    gu3 = gu.reshape(B // D, D, D)
    gi3 = gi.reshape(B // D, D, D)
    out = pl.pallas_call(
        _dot_body,
        grid=(B // D // TB,),
        in_specs=[
            pl.BlockSpec((TB, D, D), lambda i: (i, 0, 0)),
            pl.BlockSpec((TB, D, D), lambda i: (i, 0, 0)),
        ],
        out_specs=pl.BlockSpec((TB, D), lambda i: (i, 0)),
        out_shape=jax.ShapeDtypeStruct((B // D, D), jnp.float32),
    )(gu3, gi3)
    return out.reshape(B)
